# baseline (speedup 1.0000x reference)
import functools

import jax
import jax.numpy as jnp
from jax import lax
from jax.experimental import pallas as pl
from jax.experimental.pallas import tpu as pltpu
from jax.experimental.pallas import tpu_sc as plsc

E = 64
D = 768
F = 1024
T = 2048
C = max(int(round(1.0 * T / E)), 4)
SLOTS = E * C

TB = 256

NC = 2
NS = 16
L = 16
HALF = SLOTS // NC
SPT = HALF // NS
TPW = T // (NC * NS)


def _routing_body(x_ref, wr_ref, slot_ref, gate_ref, carry_ref):
    i = pl.program_id(0)

    @pl.when(i == 0)
    def _():
        carry_ref[...] = jnp.zeros_like(carry_ref)

    x = x_ref[...]
    logits = jnp.dot(x, wr_ref[...], preferred_element_type=jnp.float32)
    m = jnp.max(logits, axis=1, keepdims=True)
    s = jnp.sum(jnp.exp(logits - m), axis=1, keepdims=True)
    gate = 1.0 / s

    iota_e = lax.broadcasted_iota(jnp.int32, (TB, E), 1)
    is_max = logits == m
    e_idx = jnp.min(jnp.where(is_max, iota_e, E), axis=1, keepdims=True)
    oh = (iota_e == e_idx).astype(jnp.float32)

    r = lax.broadcasted_iota(jnp.int32, (TB, TB), 0)
    c = lax.broadcasted_iota(jnp.int32, (TB, TB), 1)
    tri = (c <= r).astype(jnp.float32)
    prefix = jnp.dot(tri, oh, preferred_element_type=jnp.float32) + carry_ref[...]
    carry_ref[...] = carry_ref[...] + jnp.sum(oh, axis=0, keepdims=True)

    pos = jnp.sum(prefix * oh, axis=1, keepdims=True) - 1.0
    valid = (pos < C).astype(jnp.float32)
    posc = jnp.minimum(pos, C - 1).astype(jnp.int32)
    slot_ref[...] = e_idx * C + posc
    gate_ref[...] = gate * valid


def _routing(x, w_router):
    return pl.pallas_call(
        _routing_body,
        grid=(T // TB,),
        in_specs=[
            pl.BlockSpec((TB, D), lambda i: (i, 0)),
            pl.BlockSpec((D, E), lambda i: (0, 0)),
        ],
        out_specs=[
            pl.BlockSpec((TB, 1), lambda i: (i, 0)),
            pl.BlockSpec((TB, 1), lambda i: (i, 0)),
        ],
        out_shape=[
            jax.ShapeDtypeStruct((T, 1), jnp.int32),
            jax.ShapeDtypeStruct((T, 1), jnp.float32),
        ],
        scratch_shapes=[pltpu.VMEM((1, E), jnp.float32)],
    )(x, w_router)


def _dispatch_body(slot_hbm, gate_hbm, x_hbm, ei_hbm,
                   slot_v, gate_v, tos_sh, tos_v, idx_v, rows_v, sem):
    cid = lax.axis_index("c")
    sid = lax.axis_index("s")

    @pl.when(sid == 0)
    def _():
        pltpu.sync_copy(slot_hbm, slot_v)
        pltpu.sync_copy(gate_hbm, gate_v)
        lo = cid * HALF

        def zero_body(i, _):
            tos_v[pl.ds(i * L, L)] = jnp.zeros((L,), jnp.int32)
            return 0

        lax.fori_loop(0, HALF // L, zero_body, 0)

        def scat_body(i, _):
            sv = slot_v[pl.ds(i * L, L)]
            gv = gate_v[pl.ds(i * L, L)]
            tok = lax.iota(jnp.int32, L) + i * L
            lsv = sv - lo
            mask = (gv > 0.0) & (lsv >= 0) & (lsv < HALF)
            lsv = jnp.clip(lsv, 0, HALF - 1)
            plsc.store_scatter(tos_v, [lsv], tok, mask=mask)
            return 0

        lax.fori_loop(0, T // L, scat_body, 0)
        pltpu.sync_copy(tos_v, tos_sh)

    plsc.subcore_barrier()
    pltpu.sync_copy(tos_sh.at[pl.ds(sid * SPT, SPT)], idx_v)
    pltpu.async_copy(x_hbm.at[idx_v], rows_v, sem).wait()
    pltpu.sync_copy(rows_v, ei_hbm.at[pl.ds(cid * HALF + sid * SPT, SPT)])


def _dispatch(slot, gate, x):
    mesh = plsc.VectorSubcoreMesh(
        core_axis_name="c", subcore_axis_name="s", num_cores=NC, num_subcores=NS)
    return pl.kernel(
        _dispatch_body,
        out_type=jax.ShapeDtypeStruct((SLOTS, D), jnp.float32),
        mesh=mesh,
        compiler_params=pltpu.CompilerParams(needs_layout_passes=False),
        scratch_types=[
            pltpu.VMEM((T,), jnp.int32),
            pltpu.VMEM((T,), jnp.float32),
            pltpu.VMEM_SHARED((HALF,), jnp.int32),
            pltpu.VMEM((HALF,), jnp.int32),
            pltpu.VMEM((SPT,), jnp.int32),
            pltpu.VMEM((SPT, D), jnp.float32),
            pltpu.SemaphoreType.DMA,
        ],
    )(slot, gate, x)


def _ffn_body(ei_ref, w1_ref, b1_ref, w2_ref, b2_ref, eo_ref):
    a = ei_ref[0]
    h = jnp.dot(a, w1_ref[0], preferred_element_type=jnp.float32) + b1_ref[0]
    h = jnp.maximum(h, 0.0)
    o = jnp.dot(h, w2_ref[0], preferred_element_type=jnp.float32) + b2_ref[0]
    eo_ref[0] = o


def _ffn(ei, w1, b1, w2, b2):
    return pl.pallas_call(
        _ffn_body,
        grid=(E,),
        in_specs=[
            pl.BlockSpec((1, C, D), lambda e: (e, 0, 0)),
            pl.BlockSpec((1, D, F), lambda e: (e, 0, 0)),
            pl.BlockSpec((1, 1, F), lambda e: (e, 0, 0)),
            pl.BlockSpec((1, F, D), lambda e: (e, 0, 0)),
            pl.BlockSpec((1, 1, D), lambda e: (e, 0, 0)),
        ],
        out_specs=pl.BlockSpec((1, C, D), lambda e: (e, 0, 0)),
        out_shape=jax.ShapeDtypeStruct((E, C, D), jnp.float32),
    )(ei, w1, b1, w2, b2)


def _combine_body(slot_hbm, eo_hbm, y_hbm, idx_v, rows_v, sem):
    cid = lax.axis_index("c")
    sid = lax.axis_index("s")
    base = (sid * NC + cid) * TPW
    pltpu.sync_copy(slot_hbm.at[pl.ds(base, TPW)], idx_v)
    pltpu.async_copy(eo_hbm.at[idx_v], rows_v, sem).wait()
    pltpu.sync_copy(rows_v, y_hbm.at[pl.ds(base, TPW)])


def _combine(slot, eo):
    mesh = plsc.VectorSubcoreMesh(
        core_axis_name="c", subcore_axis_name="s", num_cores=NC, num_subcores=NS)
    return pl.kernel(
        _combine_body,
        out_type=jax.ShapeDtypeStruct((T, D), jnp.float32),
        mesh=mesh,
        compiler_params=pltpu.CompilerParams(needs_layout_passes=False),
        scratch_types=[
            pltpu.VMEM((TPW,), jnp.int32),
            pltpu.VMEM((TPW, D), jnp.float32),
            pltpu.SemaphoreType.DMA,
        ],
    )(slot, eo)


def _scale_body(y_ref, g_ref, o_ref):
    o_ref[...] = y_ref[...] * g_ref[...]


def _scale(y, gate):
    return pl.pallas_call(
        _scale_body,
        grid=(T // TB,),
        in_specs=[
            pl.BlockSpec((TB, D), lambda i: (i, 0)),
            pl.BlockSpec((TB, 1), lambda i: (i, 0)),
        ],
        out_specs=pl.BlockSpec((TB, D), lambda i: (i, 0)),
        out_shape=jax.ShapeDtypeStruct((T, D), jnp.float32),
    )(y, gate)


def kernel(inputs, W_router, W1, b1, W2, b2):
    Bv, Sv, d = inputs.shape
    x = inputs.reshape(T, D)
    slot2, gate2 = _routing(x, W_router)
    slot = slot2.reshape(T)
    gate = gate2.reshape(T)
    ei = _dispatch(slot, gate, x)
    eo = _ffn(ei.reshape(E, C, D), W1, b1.reshape(E, 1, F),
              W2, b2.reshape(E, 1, D))
    y = _combine(slot, eo.reshape(SLOTS, D))
    out = _scale(y, gate2)
    return out.reshape(Bv, Sv, d)
